# Initial kernel scaffold; baseline (speedup 1.0000x reference)
import jax
import jax.numpy as jnp
from jax.experimental import pallas as pl


def kernel(x, edge_index, edge_w, rel_w0, loop_w0, bias0, gamma0, beta0, rel_w1, loop_w1, bias1, gamma1, beta1, rel_w2, loop_w2, bias2, gamma2, beta2, lin_w, lin_b, gamma_l, beta_l):
    raise NotImplementedError("write your pallas kernel here")


# speedup vs baseline: 21.8861x; 21.8861x over previous
import functools

import jax
import jax.numpy as jnp
from jax import lax
from jax.experimental import pallas as pl
from jax.experimental.pallas import tpu as pltpu
from jax.experimental.pallas import tpu_sc as plsc

N = 10000
D = 128
E = 320000
R = 4
EPS = 1e-5

NC = 2
NS = 16
NPAD = 10240
ROWS_PER_TILE = NPAD // NS
EPT = E // (NC * NS)
CH = 80
NCHUNK = EPT // CH
ZROWS = 40

BN_ROWS = 2000
GRID = N // BN_ROWS


def _etype_body(src_ref, w0_ref, w1_ref, w2_ref, w3_ref, gidx_ref):
    w0, w1, w2, w3 = w0_ref[...], w1_ref[...], w2_ref[...], w3_ref[...]
    et = jnp.where(w1 > w0, 1, 0)
    mx = jnp.maximum(w0, w1)
    et = jnp.where(w2 > mx, 2, et)
    mx = jnp.maximum(mx, w2)
    et = jnp.where(w3 > mx, 3, et)
    gidx_ref[...] = et.astype(jnp.int32) * N + src_ref[...]


def _gather_indices(src2, ews):
    return pl.pallas_call(
        _etype_body,
        out_shape=jax.ShapeDtypeStruct(src2.shape, jnp.int32),
    )(src2, *ews)


def _hr1_body(x_ref, w_ref, hr_ref):
    xb = x_ref[...]
    for r in range(R):
        hr_ref[r] = jnp.dot(xb, w_ref[r], preferred_element_type=jnp.float32)


def _hr_first(x, rel_w):
    return pl.pallas_call(
        _hr1_body,
        grid=(GRID,),
        in_specs=[pl.BlockSpec((BN_ROWS, D), lambda i: (i, 0)),
                  pl.BlockSpec((R, D, D), lambda i: (0, 0, 0))],
        out_specs=pl.BlockSpec((R, BN_ROWS, D), lambda i: (0, i, 0)),
        out_shape=jax.ShapeDtypeStruct((R, N, D), jnp.float32),
    )(x, rel_w)


def _hr_bn_body(pre_ref, scale_ref, shift_ref, w_ref, h_ref, hr_ref):
    h = jnp.maximum(pre_ref[...] * scale_ref[...] + shift_ref[...], 0.0)
    h_ref[...] = h
    for r in range(R):
        hr_ref[r] = jnp.dot(h, w_ref[r], preferred_element_type=jnp.float32)


def _hr_next(pre, scale, shift, rel_w):
    return pl.pallas_call(
        _hr_bn_body,
        grid=(GRID,),
        in_specs=[pl.BlockSpec((BN_ROWS, D), lambda i: (i, 0)),
                  pl.BlockSpec((1, D), lambda i: (0, 0)),
                  pl.BlockSpec((1, D), lambda i: (0, 0)),
                  pl.BlockSpec((R, D, D), lambda i: (0, 0, 0))],
        out_specs=[pl.BlockSpec((BN_ROWS, D), lambda i: (i, 0)),
                   pl.BlockSpec((R, BN_ROWS, D), lambda i: (0, i, 0))],
        out_shape=[jax.ShapeDtypeStruct((N, D), jnp.float32),
                   jax.ShapeDtypeStruct((R, N, D), jnp.float32)],
    )(pre, scale, shift, rel_w)


def _pre_body(agg_ref, x_ref, lw_ref, b_ref, g_ref, bb_ref,
              pre_ref, scale_ref, shift_ref, ssum, ssq):
    i = pl.program_id(0)

    @pl.when(i == 0)
    def _():
        ssum[...] = jnp.zeros_like(ssum)
        ssq[...] = jnp.zeros_like(ssq)

    pre = (agg_ref[0] + agg_ref[1]
           + jnp.dot(x_ref[...], lw_ref[...], preferred_element_type=jnp.float32)
           + b_ref[...])
    pre_ref[...] = pre
    ssum[...] += jnp.sum(pre, axis=0, keepdims=True)
    ssq[...] += jnp.sum(pre * pre, axis=0, keepdims=True)

    @pl.when(i == GRID - 1)
    def _():
        mean = ssum[...] * (1.0 / N)
        var = ssq[...] * (1.0 / N) - mean * mean
        sc = g_ref[...] * lax.rsqrt(var + EPS)
        scale_ref[...] = sc
        shift_ref[...] = bb_ref[...] - mean * sc


def _pre_stats(agg, x, loop_w, bias, gamma, beta):
    return pl.pallas_call(
        _pre_body,
        grid=(GRID,),
        in_specs=[pl.BlockSpec((NC, BN_ROWS, D), lambda i: (0, i, 0)),
                  pl.BlockSpec((BN_ROWS, D), lambda i: (i, 0)),
                  pl.BlockSpec((D, D), lambda i: (0, 0)),
                  pl.BlockSpec((1, D), lambda i: (0, 0)),
                  pl.BlockSpec((1, D), lambda i: (0, 0)),
                  pl.BlockSpec((1, D), lambda i: (0, 0))],
        out_specs=[pl.BlockSpec((BN_ROWS, D), lambda i: (i, 0)),
                   pl.BlockSpec((1, D), lambda i: (0, 0)),
                   pl.BlockSpec((1, D), lambda i: (0, 0))],
        out_shape=[jax.ShapeDtypeStruct((N, D), jnp.float32),
                   jax.ShapeDtypeStruct((1, D), jnp.float32),
                   jax.ShapeDtypeStruct((1, D), jnp.float32)],
        scratch_shapes=[pltpu.VMEM((1, D), jnp.float32),
                        pltpu.VMEM((1, D), jnp.float32)],
    )(agg, x, loop_w, bias, gamma, beta)


def _fin_body(pre_ref, scale_ref, shift_ref, wT_ref, b_ref, g_ref, bb_ref,
              out_ref, scl_ref, shf_ref, ssum, ssq):
    i = pl.program_id(0)

    @pl.when(i == 0)
    def _():
        ssum[...] = jnp.zeros_like(ssum)
        ssq[...] = jnp.zeros_like(ssq)

    h = jnp.maximum(pre_ref[...] * scale_ref[...] + shift_ref[...], 0.0)
    op = jnp.dot(h, wT_ref[...], preferred_element_type=jnp.float32) + b_ref[...]
    out_ref[...] = op
    ssum[...] += jnp.sum(op, axis=0, keepdims=True)
    ssq[...] += jnp.sum(op * op, axis=0, keepdims=True)

    @pl.when(i == GRID - 1)
    def _():
        mean = ssum[...] * (1.0 / N)
        var = ssq[...] * (1.0 / N) - mean * mean
        sc = g_ref[...] * lax.rsqrt(var + EPS)
        scl_ref[...] = sc
        shf_ref[...] = bb_ref[...] - mean * sc


def _final_linear(pre, scale, shift, lin_wT, lin_b, gamma_l, beta_l):
    return pl.pallas_call(
        _fin_body,
        grid=(GRID,),
        in_specs=[pl.BlockSpec((BN_ROWS, D), lambda i: (i, 0)),
                  pl.BlockSpec((1, D), lambda i: (0, 0)),
                  pl.BlockSpec((1, D), lambda i: (0, 0)),
                  pl.BlockSpec((D, D), lambda i: (0, 0)),
                  pl.BlockSpec((1, D), lambda i: (0, 0)),
                  pl.BlockSpec((1, D), lambda i: (0, 0)),
                  pl.BlockSpec((1, D), lambda i: (0, 0))],
        out_specs=[pl.BlockSpec((BN_ROWS, D), lambda i: (i, 0)),
                   pl.BlockSpec((1, D), lambda i: (0, 0)),
                   pl.BlockSpec((1, D), lambda i: (0, 0))],
        out_shape=[jax.ShapeDtypeStruct((N, D), jnp.float32),
                   jax.ShapeDtypeStruct((1, D), jnp.float32),
                   jax.ShapeDtypeStruct((1, D), jnp.float32)],
        scratch_shapes=[pltpu.VMEM((1, D), jnp.float32),
                        pltpu.VMEM((1, D), jnp.float32)],
    )(pre, scale, shift, lin_wT, lin_b, gamma_l, beta_l)


def _norm_body(x_ref, scale_ref, shift_ref, out_ref):
    out_ref[...] = x_ref[...] * scale_ref[...] + shift_ref[...]


def _normalize(x, scale, shift):
    return pl.pallas_call(
        _norm_body,
        grid=(GRID,),
        in_specs=[pl.BlockSpec((BN_ROWS, D), lambda i: (i, 0)),
                  pl.BlockSpec((1, D), lambda i: (0, 0)),
                  pl.BlockSpec((1, D), lambda i: (0, 0))],
        out_specs=pl.BlockSpec((BN_ROWS, D), lambda i: (i, 0)),
        out_shape=jax.ShapeDtypeStruct((N, D), jnp.float32),
    )(x, scale, shift)


def _sc_edge_agg(hr2, gidx4, didx4):
    mesh = plsc.VectorSubcoreMesh(core_axis_name="c", subcore_axis_name="s")

    @functools.partial(
        pl.kernel,
        out_type=jax.ShapeDtypeStruct((NC, NPAD, D), jnp.float32),
        mesh=mesh,
        scratch_types=[
            pltpu.VMEM((NCHUNK, CH), jnp.int32),
            pltpu.VMEM((NCHUNK, CH), jnp.int32),
            pltpu.VMEM((CH, D), jnp.float32),
            pltpu.VMEM((ZROWS, D), jnp.float32),
            pltpu.VMEM_SHARED((NPAD, D), jnp.float32),
            pltpu.SemaphoreType.DMA,
        ],
    )
    def k(hr_hbm, gidx_hbm, didx_hbm, out_hbm,
          gidx_v, didx_v, rows_v, zbuf, agg_sh, sem):
        c = lax.axis_index("c")
        s = lax.axis_index("s")
        row0 = s * ROWS_PER_TILE

        def zfill(i, carry):
            zbuf[i // 8, pl.ds((i % 8) * 16, 16)] = jnp.zeros((16,), jnp.float32)
            return carry
        lax.fori_loop(0, ZROWS * 8, zfill, 0)

        def zcopy(kk, carry):
            pltpu.sync_copy(zbuf, agg_sh.at[pl.ds(row0 + kk * ZROWS, ZROWS), :])
            return carry
        lax.fori_loop(0, ROWS_PER_TILE // ZROWS, zcopy, 0)

        pltpu.sync_copy(gidx_hbm.at[c, s], gidx_v)
        pltpu.sync_copy(didx_hbm.at[c, s], didx_v)
        plsc.subcore_barrier()

        def body(j, carry):
            pltpu.async_copy(hr_hbm.at[gidx_v.at[j]], rows_v, sem).wait()
            pltpu.sync_copy(rows_v, agg_sh.at[didx_v.at[j]], add=True)
            return carry
        lax.fori_loop(0, NCHUNK, body, 0)

        plsc.subcore_barrier()
        pltpu.sync_copy(agg_sh.at[pl.ds(row0, ROWS_PER_TILE), :],
                        out_hbm.at[c, pl.ds(row0, ROWS_PER_TILE), :])

    return k(hr2, gidx4, didx4)


def kernel(x, edge_index, edge_w,
           rel_w0, loop_w0, bias0, gamma0, beta0,
           rel_w1, loop_w1, bias1, gamma1, beta1,
           rel_w2, loop_w2, bias2, gamma2, beta2,
           lin_w, lin_b, gamma_l, beta_l):
    src2 = edge_index[0].reshape(E // D, D)
    ews = [edge_w[:, r].reshape(E // D, D) for r in range(R)]
    gidx4 = _gather_indices(src2, ews).reshape(NC, NS, NCHUNK, CH)
    didx4 = edge_index[1].reshape(NC, NS, NCHUNK, CH)

    rel_ws = (rel_w0, rel_w1, rel_w2)
    loop_ws = (loop_w0, loop_w1, loop_w2)
    biases = (bias0.reshape(1, D), bias1.reshape(1, D), bias2.reshape(1, D))
    gammas = (gamma0.reshape(1, D), gamma1.reshape(1, D), gamma2.reshape(1, D))
    betas = (beta0.reshape(1, D), beta1.reshape(1, D), beta2.reshape(1, D))

    h = x
    pre = scale = shift = None
    for l in range(3):
        if l == 0:
            hr = _hr_first(x, rel_ws[0])
        else:
            h, hr = _hr_next(pre, scale, shift, rel_ws[l])
        agg = _sc_edge_agg(hr.reshape(R * N, D), gidx4, didx4)[:, :N, :]
        pre, scale, shift = _pre_stats(agg, h, loop_ws[l], biases[l],
                                       gammas[l], betas[l])

    outpre, scl, shf = _final_linear(pre, scale, shift, lin_w.T,
                                     lin_b.reshape(1, D),
                                     gamma_l.reshape(1, D),
                                     beta_l.reshape(1, D))
    return _normalize(outpre, scl, shf)
